# baseline (speedup 1.0000x reference)
import functools

import jax
import jax.numpy as jnp
from jax import lax
from jax.experimental import pallas as pl
from jax.experimental.pallas import tpu as pltpu
from jax.experimental.pallas import tpu_sc as plsc

DIM = 128
BATCH = 16384

_INFO = plsc.get_sparse_core_info()
_NC, _NS, _L = _INFO.num_cores, _INFO.num_subcores, _INFO.num_lanes
_NW = _NC * _NS
_BPW = BATCH // _NW
_C = 64
_NCHUNK = _BPW // _C
_GROUPS = _C // _L
_NBUF = 4
_SUPER = _NCHUNK // _NBUF


def _sc_body(z_hbm, hidx_hbm, ridx_hbm, tidx_hbm, rel_hbm, out_hbm,
             idx_all, row_bufs, sems, stage_v, out_v):
    wid = lax.axis_index("s") * _NC + lax.axis_index("c")
    base = wid * _BPW

    lane = lax.broadcasted_iota(jnp.int32, (_L,), 0)
    lane16 = lane * _L

    hidx_v, ridx_v, tidx_v = idx_all
    pltpu.sync_copy(hidx_hbm.at[pl.ds(base, _BPW)], hidx_v)
    pltpu.sync_copy(ridx_hbm.at[pl.ds(base, _BPW)], ridx_v)
    pltpu.sync_copy(tidx_hbm.at[pl.ds(base, _BPW)], tidx_v)

    def issue(chunk, p):
        hrows_v, wrows_v, trows_v = row_bufs[p]
        off = chunk * _C
        pltpu.async_copy(z_hbm.at[hidx_v.at[pl.ds(off, _C)]], hrows_v, sems[p])
        pltpu.async_copy(rel_hbm.at[ridx_v.at[pl.ds(off, _C)]], wrows_v, sems[p])
        pltpu.async_copy(z_hbm.at[tidx_v.at[pl.ds(off, _C)]], trows_v, sems[p])

    for p in range(_NBUF):
        issue(p, p)

    def super_body(sc, carry):
        for j in range(_NBUF):
            chunk = sc * _NBUF + j
            hrows_v, wrows_v, trows_v = row_bufs[j]
            pltpu.make_async_copy(z_hbm.at[hidx_v.at[pl.ds(0, _C)]],
                                  hrows_v, sems[j]).wait()
            pltpu.make_async_copy(rel_hbm.at[ridx_v.at[pl.ds(0, _C)]],
                                  wrows_v, sems[j]).wait()
            pltpu.make_async_copy(z_hbm.at[tidx_v.at[pl.ds(0, _C)]],
                                  trows_v, sems[j]).wait()

            def group_body(g, gcarry):
                @plsc.parallel_loop(0, _L, 1, unroll=8)
                def tri_body(i):
                    row = g * _L + i
                    acc = jnp.zeros((_L,), jnp.float32)
                    for s in range(DIM // _L):
                        hv = hrows_v[row, pl.ds(s * _L, _L)]
                        wv = wrows_v[row, pl.ds(s * _L, _L)]
                        tv = trows_v[row, pl.ds(s * _L, _L)]
                        acc = acc + hv * wv * tv
                    stage_v[pl.ds(i * _L, _L)] = acc
                tot = jnp.zeros((_L,), jnp.float32)
                for jj in range(_L):
                    tot = tot + plsc.load_gather(stage_v, [lane16 + jj])
                out_v[pl.ds(chunk * _C + g * _L, _L)] = tot
                return gcarry

            lax.fori_loop(0, _GROUPS, group_body, 0)

            @pl.when(sc + 1 < _SUPER)
            def _():
                issue(chunk + _NBUF, j)

        return carry

    lax.fori_loop(0, _SUPER, super_body, 0)

    pltpu.sync_copy(out_v, out_hbm.at[pl.ds(base, _BPW)])


def _flat_body(z_hbm, hidx_hbm, ridx_hbm, tidx_hbm, rel_hbm, out_hbm,
               ih, ir, it,
               r0h, r0w, r0t, r1h, r1w, r1t, r2h, r2w, r2t, r3h, r3w, r3t,
               sem0, sem1, sem2, sem3, stage_v, out_v):
    row_bufs = [(r0h, r0w, r0t), (r1h, r1w, r1t),
                (r2h, r2w, r2t), (r3h, r3w, r3t)]
    _sc_body(z_hbm, hidx_hbm, ridx_hbm, tidx_hbm, rel_hbm, out_hbm,
             (ih, ir, it), row_bufs, [sem0, sem1, sem2, sem3],
             stage_v, out_v)


@jax.jit
def _dist_mult_sc(z, h, r, t, rel_weight):
    mesh = plsc.VectorSubcoreMesh(core_axis_name="c", subcore_axis_name="s")
    f = functools.partial(
        pl.kernel,
        mesh=mesh,
        out_type=jax.ShapeDtypeStruct((BATCH,), jnp.float32),
        scratch_types=(
            [pltpu.VMEM((_BPW,), jnp.int32)] * 3
            + [pltpu.VMEM((_C, DIM), jnp.float32)] * 12
            + [pltpu.SemaphoreType.DMA] * 4
            + [pltpu.VMEM((_L * _L,), jnp.float32),
               pltpu.VMEM((_BPW,), jnp.float32)]
        ),
        compiler_params=pltpu.CompilerParams(needs_layout_passes=False),
    )(_flat_body)
    return f(z, h, r, t, rel_weight)


def kernel(z, triples, rel_weight):
    tri = triples.astype(jnp.int32)
    return _dist_mult_sc(z, tri[:, 0], tri[:, 1], tri[:, 2], rel_weight)


# speedup vs baseline: 1.1191x; 1.1191x over previous
import functools

import jax
import jax.numpy as jnp
from jax import lax
from jax.experimental import pallas as pl
from jax.experimental.pallas import tpu as pltpu
from jax.experimental.pallas import tpu_sc as plsc

DIM = 128
BATCH = 16384

_INFO = plsc.get_sparse_core_info()
_NC, _NS, _L = _INFO.num_cores, _INFO.num_subcores, _INFO.num_lanes
_NW = _NC * _NS
_BPW = BATCH // _NW
_C = 128
_NCHUNK = _BPW // _C
_GROUPS = _C // _L
_NBUF = 2
_SUPER = _NCHUNK // _NBUF


def _sc_body(z_hbm, hidx_hbm, ridx_hbm, tidx_hbm, rel_hbm, out_hbm,
             idx_all, row_bufs, sems, stage_v, out_v):
    wid = lax.axis_index("s") * _NC + lax.axis_index("c")
    base = wid * _BPW

    lane = lax.broadcasted_iota(jnp.int32, (_L,), 0)
    lane16 = lane * _L

    hidx_v, ridx_v, tidx_v = idx_all
    pltpu.sync_copy(hidx_hbm.at[pl.ds(base, _BPW)], hidx_v)
    pltpu.sync_copy(ridx_hbm.at[pl.ds(base, _BPW)], ridx_v)
    pltpu.sync_copy(tidx_hbm.at[pl.ds(base, _BPW)], tidx_v)

    def issue(chunk, p):
        hrows_v, wrows_v, trows_v = row_bufs[p]
        off = chunk * _C
        pltpu.async_copy(z_hbm.at[hidx_v.at[pl.ds(off, _C)]], hrows_v, sems[p])
        pltpu.async_copy(rel_hbm.at[ridx_v.at[pl.ds(off, _C)]], wrows_v, sems[p])
        pltpu.async_copy(z_hbm.at[tidx_v.at[pl.ds(off, _C)]], trows_v, sems[p])

    for p in range(_NBUF):
        issue(p, p)

    def super_body(sc, carry):
        for j in range(_NBUF):
            chunk = sc * _NBUF + j
            hrows_v, wrows_v, trows_v = row_bufs[j]
            pltpu.make_async_copy(z_hbm.at[hidx_v.at[pl.ds(0, _C)]],
                                  hrows_v, sems[j]).wait()
            pltpu.make_async_copy(rel_hbm.at[ridx_v.at[pl.ds(0, _C)]],
                                  wrows_v, sems[j]).wait()
            pltpu.make_async_copy(z_hbm.at[tidx_v.at[pl.ds(0, _C)]],
                                  trows_v, sems[j]).wait()

            def group_body(g, gcarry):
                @plsc.parallel_loop(0, _L, 1, unroll=4)
                def tri_body(i):
                    row = g * _L + i
                    acc = jnp.zeros((_L,), jnp.float32)
                    for s in range(DIM // _L):
                        hv = hrows_v[row, pl.ds(s * _L, _L)]
                        wv = wrows_v[row, pl.ds(s * _L, _L)]
                        tv = trows_v[row, pl.ds(s * _L, _L)]
                        acc = acc + hv * wv * tv
                    stage_v[pl.ds(i * _L, _L)] = acc
                tot = jnp.zeros((_L,), jnp.float32)
                for jj in range(_L):
                    tot = tot + plsc.load_gather(stage_v, [lane16 + jj])
                out_v[pl.ds(chunk * _C + g * _L, _L)] = tot
                return gcarry

            lax.fori_loop(0, _GROUPS, group_body, 0)

            @pl.when(sc + 1 < _SUPER)
            def _():
                issue(chunk + _NBUF, j)

        return carry

    lax.fori_loop(0, _SUPER, super_body, 0)

    pltpu.sync_copy(out_v, out_hbm.at[pl.ds(base, _BPW)])


def _flat_body(z_hbm, hidx_hbm, ridx_hbm, tidx_hbm, rel_hbm, out_hbm,
               ih, ir, it,
               r0h, r0w, r0t, r1h, r1w, r1t,
               sem0, sem1, stage_v, out_v):
    row_bufs = [(r0h, r0w, r0t), (r1h, r1w, r1t)]
    _sc_body(z_hbm, hidx_hbm, ridx_hbm, tidx_hbm, rel_hbm, out_hbm,
             (ih, ir, it), row_bufs, [sem0, sem1],
             stage_v, out_v)


@jax.jit
def _dist_mult_sc(z, h, r, t, rel_weight):
    mesh = plsc.VectorSubcoreMesh(core_axis_name="c", subcore_axis_name="s")
    f = functools.partial(
        pl.kernel,
        mesh=mesh,
        out_type=jax.ShapeDtypeStruct((BATCH,), jnp.float32),
        scratch_types=(
            [pltpu.VMEM((_BPW,), jnp.int32)] * 3
            + [pltpu.VMEM((_C, DIM), jnp.float32)] * 6
            + [pltpu.SemaphoreType.DMA] * 2
            + [pltpu.VMEM((_L * _L,), jnp.float32),
               pltpu.VMEM((_BPW,), jnp.float32)]
        ),
        compiler_params=pltpu.CompilerParams(needs_layout_passes=False),
    )(_flat_body)
    return f(z, h, r, t, rel_weight)


def kernel(z, triples, rel_weight):
    tri = triples.astype(jnp.int32)
    return _dist_mult_sc(z, tri[:, 0], tri[:, 1], tri[:, 2], rel_weight)


# speedup vs baseline: 1.1572x; 1.0340x over previous
import functools

import jax
import jax.numpy as jnp
from jax import lax
from jax.experimental import pallas as pl
from jax.experimental.pallas import tpu as pltpu
from jax.experimental.pallas import tpu_sc as plsc

DIM = 128
BATCH = 16384

_INFO = plsc.get_sparse_core_info()
_NC, _NS, _L = _INFO.num_cores, _INFO.num_subcores, _INFO.num_lanes
_NW = _NC * _NS
_BPW = BATCH // _NW
_C = 128
_NCHUNK = _BPW // _C
_GROUPS = _C // _L
_NBUF = 2
_SUPER = _NCHUNK // _NBUF


def _sc_body(z_hbm, hidx_hbm, ridx_hbm, tidx_hbm, rel_hbm, out_hbm,
             idx_all, row_bufs, sems, stage_v, out_v):
    wid = lax.axis_index("s") * _NC + lax.axis_index("c")
    base = wid * _BPW

    lane = lax.broadcasted_iota(jnp.int32, (_L,), 0)
    lane16 = lane * _L

    hidx_v, ridx_v, tidx_v = idx_all
    ih = pltpu.async_copy(hidx_hbm.at[pl.ds(base, _BPW)], hidx_v, sems[0])
    ir = pltpu.async_copy(ridx_hbm.at[pl.ds(base, _BPW)], ridx_v, sems[0])
    it = pltpu.async_copy(tidx_hbm.at[pl.ds(base, _BPW)], tidx_v, sems[0])
    ih.wait()
    ir.wait()
    it.wait()

    def issue(chunk, p):
        hrows_v, wrows_v, trows_v = row_bufs[p]
        off = chunk * _C
        pltpu.async_copy(z_hbm.at[hidx_v.at[pl.ds(off, _C)]], hrows_v, sems[p])
        pltpu.async_copy(rel_hbm.at[ridx_v.at[pl.ds(off, _C)]], wrows_v, sems[p])
        pltpu.async_copy(z_hbm.at[tidx_v.at[pl.ds(off, _C)]], trows_v, sems[p])

    for p in range(_NBUF):
        issue(p, p)

    def super_body(sc, carry):
        for j in range(_NBUF):
            chunk = sc * _NBUF + j
            hrows_v, wrows_v, trows_v = row_bufs[j]
            pltpu.make_async_copy(z_hbm.at[hidx_v.at[pl.ds(0, _C)]],
                                  hrows_v, sems[j]).wait()
            pltpu.make_async_copy(rel_hbm.at[ridx_v.at[pl.ds(0, _C)]],
                                  wrows_v, sems[j]).wait()
            pltpu.make_async_copy(z_hbm.at[tidx_v.at[pl.ds(0, _C)]],
                                  trows_v, sems[j]).wait()

            def group_body(g, gcarry):
                @plsc.parallel_loop(0, _L, 1, unroll=4)
                def tri_body(i):
                    row = g * _L + i
                    acc = jnp.zeros((_L,), jnp.float32)
                    for s in range(DIM // _L):
                        hv = hrows_v[row, pl.ds(s * _L, _L)]
                        wv = wrows_v[row, pl.ds(s * _L, _L)]
                        tv = trows_v[row, pl.ds(s * _L, _L)]
                        acc = acc + hv * wv * tv
                    stage_v[pl.ds(i * _L, _L)] = acc
                tot = jnp.zeros((_L,), jnp.float32)
                for jj in range(_L):
                    tot = tot + plsc.load_gather(stage_v, [lane16 + jj])
                out_v[pl.ds(chunk * _C + g * _L, _L)] = tot
                return gcarry

            lax.fori_loop(0, _GROUPS, group_body, 0)

            @pl.when(sc + 1 < _SUPER)
            def _():
                issue(chunk + _NBUF, j)

        return carry

    lax.fori_loop(0, _SUPER, super_body, 0)

    pltpu.sync_copy(out_v, out_hbm.at[pl.ds(base, _BPW)])


def _flat_body(z_hbm, hidx_hbm, ridx_hbm, tidx_hbm, rel_hbm, out_hbm,
               ih, ir, it,
               r0h, r0w, r0t, r1h, r1w, r1t,
               sem0, sem1, stage_v, out_v):
    row_bufs = [(r0h, r0w, r0t), (r1h, r1w, r1t)]
    _sc_body(z_hbm, hidx_hbm, ridx_hbm, tidx_hbm, rel_hbm, out_hbm,
             (ih, ir, it), row_bufs, [sem0, sem1],
             stage_v, out_v)


@jax.jit
def _dist_mult_sc(z, h, r, t, rel_weight):
    mesh = plsc.VectorSubcoreMesh(core_axis_name="c", subcore_axis_name="s")
    f = functools.partial(
        pl.kernel,
        mesh=mesh,
        out_type=jax.ShapeDtypeStruct((BATCH,), jnp.float32),
        scratch_types=(
            [pltpu.VMEM((_BPW,), jnp.int32)] * 3
            + [pltpu.VMEM((_C, DIM), jnp.float32)] * 6
            + [pltpu.SemaphoreType.DMA] * 2
            + [pltpu.VMEM((_L * _L,), jnp.float32),
               pltpu.VMEM((_BPW,), jnp.float32)]
        ),
        compiler_params=pltpu.CompilerParams(needs_layout_passes=False),
    )(_flat_body)
    return f(z, h, r, t, rel_weight)


def kernel(z, triples, rel_weight):
    tri = triples.astype(jnp.int32)
    return _dist_mult_sc(z, tri[:, 0], tri[:, 1], tri[:, 2], rel_weight)
